# baseline (speedup 1.0000x reference)
import functools

import jax
import jax.numpy as jnp
from jax import lax
from jax.experimental import pallas as pl
from jax.experimental.pallas import tpu as pltpu
from jax.experimental.pallas import tpu_sc as plsc

_B, _C, _D, _H, _W = 2, 768, 32, 32, 32
_SP = _D * _H * _W
_K = 192
_CC = 256
_NW = 32
_RPW = (_B * _K) // _NW
_IPAD = 16


def _topk_body(s_ref, idx_ref):
    s = s_ref[...]
    rows = []
    for b in range(_B):
        sb = s[b]
        row = sb[None, :]
        chunks = []
        for ci in range(_C // _CC):
            scand = sb[ci * _CC:(ci + 1) * _CC][:, None]
            jj = lax.broadcasted_iota(jnp.int32, (_CC, _C), 1)
            cc = ci * _CC + lax.broadcasted_iota(jnp.int32, (_CC, _C), 0)
            beats = (row > scand) | ((row == scand) & (jj < cc))
            chunks.append(jnp.sum(beats.astype(jnp.int32), axis=1))
        rank_b = jnp.concatenate(chunks)

        kk = lax.broadcasted_iota(jnp.int32, (_K, _C), 0)
        c2 = lax.broadcasted_iota(jnp.int32, (_K, _C), 1)
        sel = (rank_b[None, :] == kk)
        rows.append(jnp.sum(jnp.where(sel, c2, 0), axis=1) + b * _C)
    idx_ref[...] = jnp.stack(rows)


_topk_call = pl.pallas_call(
    _topk_body,
    out_shape=jax.ShapeDtypeStruct((_B, _K), jnp.int32),
)


def _sc_gather_body(x_hbm, idx_hbm, out_hbm, idx_v, buf0, buf1,
                    gsem0, gsem1, ssem0, ssem1):
    wid = lax.axis_index("s") * 2 + lax.axis_index("c")
    base = wid * _RPW
    pltpu.sync_copy(idx_hbm.at[wid], idx_v)

    bufs = (buf0, buf1)
    gsems = (gsem0, gsem1)
    ssems = (ssem0, ssem1)

    def gather_start(j, buf, sem):
        return pltpu.async_copy(x_hbm.at[idx_v.at[j]], buf, sem)

    def scatter_start(j, buf, sem):
        return pltpu.async_copy(buf, out_hbm.at[pl.ds(base + j, 1)], sem)

    g = [None, None]
    s = [None, None]
    g[0] = gather_start(0, bufs[0], gsems[0])
    for j in range(_RPW):
        p = j & 1
        q = (j + 1) & 1
        if j + 1 < _RPW:
            if s[q] is not None:
                s[q].wait()
            g[q] = gather_start(j + 1, bufs[q], gsems[q])
        g[p].wait()
        s[p] = scatter_start(j, bufs[p], ssems[p])
    s[(_RPW - 2) & 1].wait()
    s[(_RPW - 1) & 1].wait()


@functools.cache
def _sc_gather_call():
    mesh = plsc.VectorSubcoreMesh(core_axis_name="c", subcore_axis_name="s")
    return pl.kernel(
        _sc_gather_body,
        out_type=jax.ShapeDtypeStruct((_B * _K, _SP), jnp.float32),
        mesh=mesh,
        scratch_types=[
            pltpu.VMEM((_IPAD, 1), jnp.int32),
            pltpu.VMEM((1, _SP), jnp.float32),
            pltpu.VMEM((1, _SP), jnp.float32),
            pltpu.SemaphoreType.DMA,
            pltpu.SemaphoreType.DMA,
            pltpu.SemaphoreType.DMA,
            pltpu.SemaphoreType.DMA,
        ],
    )


def kernel(x, W1, b1, W2, b2):
    y = jnp.mean(x, axis=(2, 3, 4))
    h = y @ W1.T + b1
    h = jnp.where(h >= 0, h, 0.01 * h)
    h = h @ W2.T + b2
    s = jax.nn.sigmoid(h)
    idx = _topk_call(s)
    idxp = (
        jnp.zeros((_NW, _IPAD), jnp.int32)
        .at[:, :_RPW].set(idx.reshape(_NW, _RPW))
        .reshape(_NW, _IPAD, 1)
    )
    out2d = _sc_gather_call()(x.reshape(_B * _C, _SP), idxp)
    return out2d.reshape(_B, _K, _D, _H, _W)
